# baseline (speedup 1.0000x reference)
import dataclasses
import functools

import jax
import jax.numpy as jnp
from jax import lax
from jax.experimental import pallas as pl
from jax.experimental.pallas import tpu as pltpu
from jax.experimental.pallas import tpu_sc as plsc

_K = 512
_N = 224 * 224
_ROWS = 392
_BLK = 56
_NW = 32
_BPW = _N // _NW


def _argmin_kernel(tab_ref, x_ref, o_ref):
    r = x_ref[0]
    g = x_ref[1]
    b = x_ref[2]

    big = jnp.full(r.shape, jnp.inf, jnp.float32)
    zeroi = jnp.zeros(r.shape, jnp.int32)

    def body(k, carry):
        best, bidx = carry
        m2r = tab_ref[k, 0]
        m2g = tab_ref[k, 1]
        m2b = tab_ref[k, 2]
        cn = tab_ref[k, 3]
        s = r * m2r + g * m2g + b * m2b + cn
        m = s < best
        best = jnp.minimum(s, best)
        bidx = jnp.where(m, k, bidx)
        return best, bidx

    _, bidx = jax.lax.fori_loop(0, _K, body, (big, zeroi), unroll=4)
    o_ref[...] = bidx


def _sc_compiler_params():
    cp = pltpu.CompilerParams()
    if "needs_layout_passes" in pltpu.CompilerParams.__dataclass_fields__:
        cp = dataclasses.replace(cp, needs_layout_passes=False)
    return cp


def _sc_gather(ctab_flat, idx_flat):
    @functools.partial(
        pl.kernel,
        compiler_params=_sc_compiler_params(),
        out_type=jax.ShapeDtypeStruct((3 * _N,), jnp.float32),
        mesh=plsc.VectorSubcoreMesh(
            core_axis_name="c", subcore_axis_name="s"),
        scratch_types=[
            pltpu.VMEM((3 * _K,), jnp.float32),
            pltpu.VMEM((_BPW,), jnp.int32),
            pltpu.VMEM((_BPW,), jnp.float32),
            pltpu.VMEM((_BPW,), jnp.float32),
            pltpu.VMEM((_BPW,), jnp.float32),
        ],
    )
    def gather_kernel(tab_hbm, i_hbm, o_hbm, tab_v, idx_v, r_v, g_v, b_v):
        wid = lax.axis_index("s") * 2 + lax.axis_index("c")
        base = wid * _BPW
        pltpu.sync_copy(tab_hbm, tab_v)
        pltpu.sync_copy(i_hbm.at[pl.ds(base, _BPW)], idx_v)

        @pl.loop(0, _BPW, step=16)
        def _(i):
            sl = pl.ds(i, 16)
            idx0 = idx_v[sl]
            r_v[sl] = plsc.load_gather(tab_v, [idx0])
            g_v[sl] = plsc.load_gather(tab_v, [idx0 + _K])
            b_v[sl] = plsc.load_gather(tab_v, [idx0 + 2 * _K])

        pltpu.sync_copy(r_v, o_hbm.at[pl.ds(base, _BPW)])
        pltpu.sync_copy(g_v, o_hbm.at[pl.ds(_N + base, _BPW)])
        pltpu.sync_copy(b_v, o_hbm.at[pl.ds(2 * _N + base, _BPW)])

    return gather_kernel(ctab_flat, idx_flat)


@jax.jit
def kernel(adv_patch, printability_colors):
    h, w = adv_patch.shape[-2], adv_patch.shape[-1]
    x = adv_patch.reshape(3, _ROWS, 128)
    c = printability_colors
    cn = jnp.sum(c * c, axis=1, keepdims=True) - 2e-6 * jnp.sum(
        c, axis=1, keepdims=True)
    tab = jnp.concatenate([-2.0 * c, cn], axis=1)

    idx = pl.pallas_call(
        _argmin_kernel,
        grid=(_ROWS // _BLK,),
        in_specs=[
            pl.BlockSpec(memory_space=pltpu.SMEM),
            pl.BlockSpec((3, _BLK, 128), lambda i: (0, i, 0)),
        ],
        out_specs=pl.BlockSpec((_BLK, 128), lambda i: (i, 0)),
        out_shape=jax.ShapeDtypeStruct((_ROWS, 128), jnp.int32),
    )(tab, x)

    ctab_flat = c.T.reshape(3 * _K)
    out = _sc_gather(ctab_flat, idx.reshape(_N))
    return out.reshape(1, 3, h, w)


# speedup vs baseline: 1.0756x; 1.0756x over previous
import dataclasses
import functools

import jax
import jax.numpy as jnp
from jax import lax
from jax.experimental import pallas as pl
from jax.experimental.pallas import tpu as pltpu
from jax.experimental.pallas import tpu_sc as plsc

_K = 512
_N = 224 * 224
_ROWS = 392
_BLK = 56
_NW = 32
_BPW = _N // _NW


def _argmin_kernel(tab_ref, x_ref, o_ref):
    r = x_ref[0]
    g = x_ref[1]
    b = x_ref[2]

    big = jnp.full(r.shape, jnp.inf, jnp.float32)
    zeroi = jnp.zeros(r.shape, jnp.int32)

    def body(k, carry):
        best, bidx = carry
        m2r = tab_ref[k, 0]
        m2g = tab_ref[k, 1]
        m2b = tab_ref[k, 2]
        cn = tab_ref[k, 3]
        s = r * m2r + g * m2g + b * m2b + cn
        m = s < best
        best = jnp.minimum(s, best)
        bidx = jnp.where(m, k, bidx)
        return best, bidx

    _, bidx = jax.lax.fori_loop(0, _K, body, (big, zeroi), unroll=32)
    o_ref[...] = bidx


def _sc_compiler_params():
    cp = pltpu.CompilerParams()
    if "needs_layout_passes" in pltpu.CompilerParams.__dataclass_fields__:
        cp = dataclasses.replace(cp, needs_layout_passes=False)
    return cp


def _sc_gather(ctab_flat, idx_flat):
    @functools.partial(
        pl.kernel,
        compiler_params=_sc_compiler_params(),
        out_type=jax.ShapeDtypeStruct((3 * _N,), jnp.float32),
        mesh=plsc.VectorSubcoreMesh(
            core_axis_name="c", subcore_axis_name="s"),
        scratch_types=[
            pltpu.VMEM((3 * _K,), jnp.float32),
            pltpu.VMEM((_BPW,), jnp.int32),
            pltpu.VMEM((_BPW,), jnp.float32),
            pltpu.VMEM((_BPW,), jnp.float32),
            pltpu.VMEM((_BPW,), jnp.float32),
        ],
    )
    def gather_kernel(tab_hbm, i_hbm, o_hbm, tab_v, idx_v, r_v, g_v, b_v):
        wid = lax.axis_index("s") * 2 + lax.axis_index("c")
        base = wid * _BPW
        pltpu.sync_copy(tab_hbm, tab_v)
        pltpu.sync_copy(i_hbm.at[pl.ds(base, _BPW)], idx_v)

        @pl.loop(0, _BPW, step=16)
        def _(i):
            sl = pl.ds(i, 16)
            idx0 = idx_v[sl]
            r_v[sl] = plsc.load_gather(tab_v, [idx0])
            g_v[sl] = plsc.load_gather(tab_v, [idx0 + _K])
            b_v[sl] = plsc.load_gather(tab_v, [idx0 + 2 * _K])

        pltpu.sync_copy(r_v, o_hbm.at[pl.ds(base, _BPW)])
        pltpu.sync_copy(g_v, o_hbm.at[pl.ds(_N + base, _BPW)])
        pltpu.sync_copy(b_v, o_hbm.at[pl.ds(2 * _N + base, _BPW)])

    return gather_kernel(ctab_flat, idx_flat)


@jax.jit
def kernel(adv_patch, printability_colors):
    h, w = adv_patch.shape[-2], adv_patch.shape[-1]
    x = adv_patch.reshape(3, _ROWS, 128)
    c = printability_colors
    cn = jnp.sum(c * c, axis=1, keepdims=True) - 2e-6 * jnp.sum(
        c, axis=1, keepdims=True)
    tab = jnp.concatenate([-2.0 * c, cn], axis=1)

    idx = pl.pallas_call(
        _argmin_kernel,
        grid=(_ROWS // _BLK,),
        in_specs=[
            pl.BlockSpec(memory_space=pltpu.SMEM),
            pl.BlockSpec((3, _BLK, 128), lambda i: (0, i, 0)),
        ],
        out_specs=pl.BlockSpec((_BLK, 128), lambda i: (i, 0)),
        out_shape=jax.ShapeDtypeStruct((_ROWS, 128), jnp.int32),
    )(tab, x)

    ctab_flat = c.T.reshape(3 * _K)
    out = _sc_gather(ctab_flat, idx.reshape(_N))
    return out.reshape(1, 3, h, w)


# speedup vs baseline: 1.1577x; 1.0763x over previous
import dataclasses
import functools

import jax
import jax.numpy as jnp
from jax import lax
from jax.experimental import pallas as pl
from jax.experimental.pallas import tpu as pltpu
from jax.experimental.pallas import tpu_sc as plsc

_K = 512
_N = 224 * 224
_ROWS = 392
_BLK = 56
_NW = 32
_BPW = _N // _NW


_G = 8
_NCHUNK = _K // 8


def _argmin_kernel(tab_ref, x_ref, o_ref):
    iota8 = jax.lax.broadcasted_iota(jnp.int32, (8, 128), 0).astype(jnp.float32)
    shape8 = (8, 128)

    def row_group(grp, _):
        base = grp * _G
        rb = [jnp.broadcast_to(x_ref[0, pl.ds(base + i, 1), :], shape8)
              for i in range(_G)]
        gb = [jnp.broadcast_to(x_ref[1, pl.ds(base + i, 1), :], shape8)
              for i in range(_G)]
        bb = [jnp.broadcast_to(x_ref[2, pl.ds(base + i, 1), :], shape8)
              for i in range(_G)]

        big = jnp.full(shape8, jnp.inf, jnp.float32)
        zero = jnp.zeros(shape8, jnp.float32)
        best = [big] * _G
        bidx = [zero] * _G

        for j in range(_NCHUNK):
            t_r = tab_ref[0, pl.ds(8 * j, 8), :]
            t_g = tab_ref[1, pl.ds(8 * j, 8), :]
            t_b = tab_ref[2, pl.ds(8 * j, 8), :]
            t_c = tab_ref[3, pl.ds(8 * j, 8), :]
            kidx = iota8 + jnp.float32(8 * j)
            for i in range(_G):
                s = rb[i] * t_r + gb[i] * t_g + bb[i] * t_b + t_c
                m = s < best[i]
                best[i] = jnp.minimum(s, best[i])
                bidx[i] = jnp.where(m, kidx, bidx[i])

        for i in range(_G):
            besti, bidxi = best[i], bidx[i]
            rmin = jnp.min(besti, axis=0, keepdims=True)
            masked = jnp.where(besti == rmin, bidxi, jnp.float32(_K))
            ridx = jnp.min(masked, axis=0, keepdims=True)
            o_ref[pl.ds(base + i, 1), :] = ridx.astype(jnp.int32)
        return 0

    jax.lax.fori_loop(0, _BLK // _G, row_group, 0)


def _sc_compiler_params():
    cp = pltpu.CompilerParams()
    if "needs_layout_passes" in pltpu.CompilerParams.__dataclass_fields__:
        cp = dataclasses.replace(cp, needs_layout_passes=False)
    return cp


def _sc_gather(ctab_flat, idx_flat):
    @functools.partial(
        pl.kernel,
        compiler_params=_sc_compiler_params(),
        out_type=jax.ShapeDtypeStruct((3 * _N,), jnp.float32),
        mesh=plsc.VectorSubcoreMesh(
            core_axis_name="c", subcore_axis_name="s"),
        scratch_types=[
            pltpu.VMEM((3 * _K,), jnp.float32),
            pltpu.VMEM((_BPW,), jnp.int32),
            pltpu.VMEM((_BPW,), jnp.float32),
            pltpu.VMEM((_BPW,), jnp.float32),
            pltpu.VMEM((_BPW,), jnp.float32),
        ],
    )
    def gather_kernel(tab_hbm, i_hbm, o_hbm, tab_v, idx_v, r_v, g_v, b_v):
        wid = lax.axis_index("s") * 2 + lax.axis_index("c")
        base = wid * _BPW
        pltpu.sync_copy(tab_hbm, tab_v)
        pltpu.sync_copy(i_hbm.at[pl.ds(base, _BPW)], idx_v)

        @pl.loop(0, _BPW, step=16)
        def _(i):
            sl = pl.ds(i, 16)
            idx0 = idx_v[sl]
            r_v[sl] = plsc.load_gather(tab_v, [idx0])
            g_v[sl] = plsc.load_gather(tab_v, [idx0 + _K])
            b_v[sl] = plsc.load_gather(tab_v, [idx0 + 2 * _K])

        pltpu.sync_copy(r_v, o_hbm.at[pl.ds(base, _BPW)])
        pltpu.sync_copy(g_v, o_hbm.at[pl.ds(_N + base, _BPW)])
        pltpu.sync_copy(b_v, o_hbm.at[pl.ds(2 * _N + base, _BPW)])

    return gather_kernel(ctab_flat, idx_flat)


@jax.jit
def kernel(adv_patch, printability_colors):
    h, w = adv_patch.shape[-2], adv_patch.shape[-1]
    x = adv_patch.reshape(3, _ROWS, 128)
    c = printability_colors
    cn = jnp.sum(c * c, axis=1, keepdims=True) - 2e-6 * jnp.sum(
        c, axis=1, keepdims=True)
    tab = jnp.concatenate([-2.0 * c, cn], axis=1)
    tabb = jnp.broadcast_to(tab.T[:, :, None], (4, _K, 128))

    idx = pl.pallas_call(
        _argmin_kernel,
        grid=(_ROWS // _BLK,),
        in_specs=[
            pl.BlockSpec((4, _K, 128), lambda i: (0, 0, 0)),
            pl.BlockSpec((3, _BLK, 128), lambda i: (0, i, 0)),
        ],
        out_specs=pl.BlockSpec((_BLK, 128), lambda i: (i, 0)),
        out_shape=jax.ShapeDtypeStruct((_ROWS, 128), jnp.int32),
    )(tabb, x)

    ctab_flat = c.T.reshape(3 * _K)
    out = _sc_gather(ctab_flat, idx.reshape(_N))
    return out.reshape(1, 3, h, w)


# speedup vs baseline: 1.1579x; 1.0001x over previous
import dataclasses
import functools

import jax
import jax.numpy as jnp
from jax import lax
from jax.experimental import pallas as pl
from jax.experimental.pallas import tpu as pltpu
from jax.experimental.pallas import tpu_sc as plsc

_K = 512
_N = 224 * 224
_ROWS = 392
_BLK = 56
_NW = 32
_BPW = _N // _NW


_G = 8
_NCHUNK = _K // 8


def _argmin_kernel(tab_ref, x_ref, o_ref):
    iota8 = jax.lax.broadcasted_iota(jnp.int32, (8, 128), 0).astype(jnp.float32)
    shape8 = (8, 128)

    def row_group(grp, _):
        base = grp * _G
        rb = [jnp.broadcast_to(x_ref[0, pl.ds(base + i, 1), :], shape8)
              for i in range(_G)]
        gb = [jnp.broadcast_to(x_ref[1, pl.ds(base + i, 1), :], shape8)
              for i in range(_G)]
        bb = [jnp.broadcast_to(x_ref[2, pl.ds(base + i, 1), :], shape8)
              for i in range(_G)]

        big = jnp.full(shape8, jnp.inf, jnp.float32)
        zero = jnp.zeros(shape8, jnp.float32)
        best = [big] * _G
        bidx = [zero] * _G

        for j in range(_NCHUNK):
            t_r = tab_ref[0, pl.ds(8 * j, 8), :]
            t_g = tab_ref[1, pl.ds(8 * j, 8), :]
            t_b = tab_ref[2, pl.ds(8 * j, 8), :]
            t_c = tab_ref[3, pl.ds(8 * j, 8), :]
            kidx = iota8 + jnp.float32(8 * j)
            for i in range(_G):
                s = rb[i] * t_r + gb[i] * t_g + bb[i] * t_b + t_c
                m = s < best[i]
                best[i] = jnp.minimum(s, best[i])
                bidx[i] = jnp.where(m, kidx, bidx[i])

        for i in range(_G):
            besti, bidxi = best[i], bidx[i]
            rmin = jnp.min(besti, axis=0, keepdims=True)
            masked = jnp.where(besti == rmin, bidxi, jnp.float32(_K))
            ridx = jnp.min(masked, axis=0, keepdims=True)
            o_ref[pl.ds(base + i, 1), :] = ridx.astype(jnp.int32)
        return 0

    jax.lax.fori_loop(0, _BLK // _G, row_group, 0)


def _sc_compiler_params():
    cp = pltpu.CompilerParams()
    if "needs_layout_passes" in pltpu.CompilerParams.__dataclass_fields__:
        cp = dataclasses.replace(cp, needs_layout_passes=False)
    return cp


def _sc_gather(ctab_flat, idx_flat):
    @functools.partial(
        pl.kernel,
        compiler_params=_sc_compiler_params(),
        out_type=jax.ShapeDtypeStruct((3 * _N,), jnp.float32),
        mesh=plsc.VectorSubcoreMesh(
            core_axis_name="c", subcore_axis_name="s"),
        scratch_types=[
            pltpu.VMEM((3 * _K,), jnp.float32),
            pltpu.VMEM((_BPW,), jnp.int32),
            pltpu.VMEM((_BPW,), jnp.float32),
            pltpu.VMEM((_BPW,), jnp.float32),
            pltpu.VMEM((_BPW,), jnp.float32),
            pltpu.SemaphoreType.DMA,
            pltpu.SemaphoreType.DMA,
        ],
    )
    def gather_kernel(tab_hbm, i_hbm, o_hbm, tab_v, idx_v, r_v, g_v, b_v,
                      sem_in, sem_out):
        wid = lax.axis_index("s") * 2 + lax.axis_index("c")
        base = wid * _BPW
        cp_tab = pltpu.make_async_copy(tab_hbm, tab_v, sem_in)
        cp_idx = pltpu.make_async_copy(i_hbm.at[pl.ds(base, _BPW)], idx_v,
                                       sem_in)
        cp_tab.start()
        cp_idx.start()
        cp_tab.wait()
        cp_idx.wait()

        half = _BPW // 2

        @pl.loop(0, half, step=16)
        def _(i):
            sl = pl.ds(i, 16)
            idx0 = idx_v[sl]
            r_v[sl] = plsc.load_gather(tab_v, [idx0])
            g_v[sl] = plsc.load_gather(tab_v, [idx0 + _K])
            b_v[sl] = plsc.load_gather(tab_v, [idx0 + 2 * _K])

        cp_r0 = pltpu.make_async_copy(
            r_v.at[pl.ds(0, half)], o_hbm.at[pl.ds(base, half)], sem_out)
        cp_g0 = pltpu.make_async_copy(
            g_v.at[pl.ds(0, half)], o_hbm.at[pl.ds(_N + base, half)], sem_out)
        cp_b0 = pltpu.make_async_copy(
            b_v.at[pl.ds(0, half)], o_hbm.at[pl.ds(2 * _N + base, half)],
            sem_out)
        cp_r0.start()
        cp_g0.start()
        cp_b0.start()

        @pl.loop(half, _BPW, step=16)
        def _(i):
            sl = pl.ds(i, 16)
            idx0 = idx_v[sl]
            r_v[sl] = plsc.load_gather(tab_v, [idx0])
            g_v[sl] = plsc.load_gather(tab_v, [idx0 + _K])
            b_v[sl] = plsc.load_gather(tab_v, [idx0 + 2 * _K])

        cp_r1 = pltpu.make_async_copy(
            r_v.at[pl.ds(half, half)], o_hbm.at[pl.ds(base + half, half)],
            sem_out)
        cp_g1 = pltpu.make_async_copy(
            g_v.at[pl.ds(half, half)],
            o_hbm.at[pl.ds(_N + base + half, half)], sem_out)
        cp_b1 = pltpu.make_async_copy(
            b_v.at[pl.ds(half, half)],
            o_hbm.at[pl.ds(2 * _N + base + half, half)], sem_out)
        cp_r1.start()
        cp_g1.start()
        cp_b1.start()
        cp_r0.wait()
        cp_g0.wait()
        cp_b0.wait()
        cp_r1.wait()
        cp_g1.wait()
        cp_b1.wait()

    return gather_kernel(ctab_flat, idx_flat)


@jax.jit
def kernel(adv_patch, printability_colors):
    h, w = adv_patch.shape[-2], adv_patch.shape[-1]
    x = adv_patch.reshape(3, _ROWS, 128)
    c = printability_colors
    cn = jnp.sum(c * c, axis=1, keepdims=True) - 2e-6 * jnp.sum(
        c, axis=1, keepdims=True)
    tab = jnp.concatenate([-2.0 * c, cn], axis=1)
    tabb = jnp.broadcast_to(tab.T[:, :, None], (4, _K, 128))

    idx = pl.pallas_call(
        _argmin_kernel,
        grid=(_ROWS // _BLK,),
        in_specs=[
            pl.BlockSpec((4, _K, 128), lambda i: (0, 0, 0)),
            pl.BlockSpec((3, _BLK, 128), lambda i: (0, i, 0)),
        ],
        out_specs=pl.BlockSpec((_BLK, 128), lambda i: (i, 0)),
        out_shape=jax.ShapeDtypeStruct((_ROWS, 128), jnp.int32),
    )(tabb, x)

    ctab_flat = c.T.reshape(3 * _K)
    out = _sc_gather(ctab_flat, idx.reshape(_N))
    return out.reshape(1, 3, h, w)
